# baseline (device time: 299484 ns/iter reference)
import numpy as np

import jax
import jax.numpy as jnp
from jax import lax
from jax.experimental import pallas as pl
from jax.experimental.pallas import tpu as pltpu

N_DEV = 4
SQ = 2048
SKV = 2048
DM = 1024
HQ_TOTAL = 32
HQ_PER = 8
DH = 128
SCALE = 0.08838834764831843

QBLK = 256
N_QB = SQ // QBLK
HP = 4
BAND = 512
GLOB = 128
NEG = -1e9
KCHUNK = 256

BF16 = jnp.bfloat16


def _kv_headmajor(k_ext, v_ext, my):

    def body(s_ref, k_in, v_in, kt_ref, vt_ref):
        kt_ref[...] = k_in[0].transpose(1, 0, 2).astype(BF16)
        vt_ref[...] = v_in[0].transpose(1, 0, 2).astype(BF16)

    grid_spec = pltpu.PrefetchScalarGridSpec(
        num_scalar_prefetch=1,
        grid=(SKV // KCHUNK, HQ_TOTAL // HQ_PER),
        in_specs=[
            pl.BlockSpec(
                (1, KCHUNK, HQ_PER, DH), lambda sk, j, s: (s[0], sk, j, 0)
            ),
            pl.BlockSpec(
                (1, KCHUNK, HQ_PER, DH), lambda sk, j, s: (s[0], sk, j, 0)
            ),
        ],
        out_specs=[
            pl.BlockSpec((HQ_PER, KCHUNK, DH), lambda sk, j, s: (j, sk, 0)),
            pl.BlockSpec((HQ_PER, KCHUNK, DH), lambda sk, j, s: (j, sk, 0)),
        ],
    )
    return pl.pallas_call(
        body,
        grid_spec=grid_spec,
        out_shape=[
            jax.ShapeDtypeStruct((HQ_TOTAL, SKV, DH), BF16),
            jax.ShapeDtypeStruct((HQ_TOTAL, SKV, DH), BF16),
        ],
        compiler_params=pltpu.CompilerParams(
            dimension_semantics=("arbitrary", "arbitrary"),
        ),
    )(jnp.reshape(my, (1,)).astype(jnp.int32), k_ext, v_ext)


def _band_mask() -> np.ndarray:
    out = np.empty((N_QB, QBLK, GLOB + BAND), np.float32)
    for qb in range(N_QB):
        kstart = int(np.clip(qb * QBLK - 128, GLOB, SKV - BAND))
        qi = (qb * QBLK + np.arange(QBLK))[:, None]
        ki = np.concatenate([np.arange(GLOB), kstart + np.arange(BAND)])[None, :]
        keep = (np.abs(qi - ki) <= 128) | (ki < 32) | (qi < 32)
        out[qb] = np.where(keep, 0.0, NEG)
    return out


def _fused(x, wq_my, wo_my, kt, vt, my):
    def body(s_ref, x_ref, wq_in, wo_in, k_ref, v_ref, mask_ref, o_ref,
             w_scr, ctx_scr, send_sems, recv_sems):
        me = s_ref[0]
        t = pl.program_id(0)
        hh = pl.program_id(1)
        qb = pl.program_id(2)
        right = lax.rem(me + 1, N_DEV)
        left = lax.rem(me + N_DEV - 1, N_DEV)
        g = lax.rem(me + N_DEV - t, N_DEV)

        @pl.when(jnp.logical_and(t == 0, jnp.logical_and(qb == 0, hh == 0)))
        def _first():
            barrier = pltpu.get_barrier_semaphore()
            for nbr in (left, right):
                pl.semaphore_signal(
                    barrier, inc=1, device_id=(nbr,),
                    device_id_type=pl.DeviceIdType.MESH,
                )
            pl.semaphore_wait(barrier, 2)
            w_scr[pl.ds(me, 1), pl.ds(0, 1)] = (
                (wq_in[...] * SCALE).reshape(1, 1, DM, DM).astype(BF16)
            )
            w_scr[pl.ds(me, 1), pl.ds(1, 1)] = (
                wo_in[...].reshape(1, 1, DM, DM).astype(BF16)
            )
            hop1 = pltpu.make_async_remote_copy(
                src_ref=w_scr.at[me],
                dst_ref=w_scr.at[me],
                send_sem=send_sems.at[0],
                recv_sem=recv_sems.at[0],
                device_id=(right,),
                device_id_type=pl.DeviceIdType.MESH,
            )
            hop1.start()

        for k in (1, 2, 3):
            @pl.when(jnp.logical_and(t == k, jnp.logical_and(qb == 0, hh == 0)))
            def _boundary(k=k):
                sent = lax.rem(me + N_DEV - (k - 1), N_DEV)
                got = lax.rem(me + N_DEV - k, N_DEV)
                prev = pltpu.make_async_remote_copy(
                    src_ref=w_scr.at[sent],
                    dst_ref=w_scr.at[got],
                    send_sem=send_sems.at[k - 1],
                    recv_sem=recv_sems.at[k - 1],
                    device_id=(right,),
                    device_id_type=pl.DeviceIdType.MESH,
                )
                prev.wait()
                if k < 3:
                    nxt = pltpu.make_async_remote_copy(
                        src_ref=w_scr.at[got],
                        dst_ref=w_scr.at[got],
                        send_sem=send_sems.at[k],
                        recv_sem=recv_sems.at[k],
                        device_id=(right,),
                        device_id_type=pl.DeviceIdType.MESH,
                    )
                    nxt.start()

        xq = x_ref[0].astype(BF16)
        wq2 = w_scr[g, 0, :, pl.ds(hh * (HP * DH), HP * DH)]
        q2 = jnp.dot(
            xq, wq2, preferred_element_type=jnp.float32
        ).astype(BF16)

        kstart = pl.multiple_of(
            jnp.clip(qb * QBLK - 128, GLOB, SKV - BAND), 128
        )
        for i in range(HP):
            qh = q2[:, i * DH:(i + 1) * DH]
            kb = k_ref[i, pl.ds(kstart, BAND), :]
            vb = v_ref[i, pl.ds(kstart, BAND), :]
            kg = k_ref[i, 0:GLOB, :]
            vg = v_ref[i, 0:GLOB, :]

            sb = lax.dot_general(
                qh, kb, (((1,), (1,)), ((), ())),
                preferred_element_type=jnp.float32,
            ) + mask_ref[0, :, GLOB:]
            sg = lax.dot_general(
                qh, kg, (((1,), (1,)), ((), ())),
                preferred_element_type=jnp.float32,
            ) + mask_ref[0, :, 0:GLOB]

            eb = jnp.exp(sb)
            eg = jnp.exp(sg)
            inv = 1.0 / (
                jnp.sum(eb, axis=-1, keepdims=True)
                + jnp.sum(eg, axis=-1, keepdims=True)
            )
            ctx = (
                jnp.dot(eb.astype(BF16), vb, preferred_element_type=jnp.float32)
                + jnp.dot(eg.astype(BF16), vg, preferred_element_type=jnp.float32)
            ) * inv
            ctx_scr[:, i * DH:(i + 1) * DH] = ctx

            @pl.when(qb == 0)
            def _glob_rows(i=i, qh=qh):
                q32 = qh[0:32]
                s32 = lax.dot_general(
                    q32, k_ref[i], (((1,), (1,)), ((), ())),
                    preferred_element_type=jnp.float32,
                )
                e32 = jnp.exp(s32)
                ctx_scr[0:32, i * DH:(i + 1) * DH] = jnp.dot(
                    e32.astype(BF16), v_ref[i],
                    preferred_element_type=jnp.float32,
                ) * (1.0 / jnp.sum(e32, axis=-1, keepdims=True))

        wo2 = w_scr[g, 1, pl.ds(hh * (HP * DH), HP * DH), :]
        contrib = jnp.dot(
            ctx_scr[...].astype(BF16), wo2, preferred_element_type=jnp.float32
        )
        o_ref[0] = contrib

    n_hh = HQ_PER // HP
    grid = (N_DEV, n_hh, N_QB)
    grid_spec = pltpu.PrefetchScalarGridSpec(
        num_scalar_prefetch=1,
        grid=grid,
        in_specs=[
            pl.BlockSpec((1, QBLK, DM), lambda t, hh, qb, s: (0, qb, 0)),
            pl.BlockSpec(memory_space=pltpu.VMEM),
            pl.BlockSpec(memory_space=pltpu.VMEM),
            pl.BlockSpec(
                (HP, SKV, DH),
                lambda t, hh, qb, s: (
                    lax.rem(s[0] + N_DEV - t, N_DEV) * n_hh + hh, 0, 0
                ),
            ),
            pl.BlockSpec(
                (HP, SKV, DH),
                lambda t, hh, qb, s: (
                    lax.rem(s[0] + N_DEV - t, N_DEV) * n_hh + hh, 0, 0
                ),
            ),
            pl.BlockSpec(
                (1, QBLK, GLOB + BAND), lambda t, hh, qb, s: (qb, 0, 0)
            ),
        ],
        out_specs=pl.BlockSpec(
            (1, QBLK, DM), lambda t, hh, qb, s: (t * n_hh + hh, qb, 0)
        ),
        scratch_shapes=[
            pltpu.VMEM((N_DEV, 2, DM, DM), BF16),
            pltpu.VMEM((QBLK, HP * DH), jnp.float32),
            pltpu.SemaphoreType.DMA((N_DEV - 1,)),
            pltpu.SemaphoreType.DMA((N_DEV - 1,)),
        ],
    )
    partials = pl.pallas_call(
        body,
        grid_spec=grid_spec,
        out_shape=jax.ShapeDtypeStruct((N_DEV * n_hh, SQ, DM), jnp.float32),
        compiler_params=pltpu.CompilerParams(
            dimension_semantics=("arbitrary", "arbitrary", "arbitrary"),
            collective_id=0,
        ),
    )(
        jnp.reshape(my, (1,)).astype(jnp.int32),
        x, wq_my, wo_my, kt, vt, jnp.asarray(_band_mask()),
    )
    return jnp.sum(partials, axis=0, keepdims=True)


def kernel(x, Wq, K_ext, V_ext, Wo):
    my = lax.axis_index("i")
    kt, vt = _kv_headmajor(K_ext, V_ext, my)
    return _fused(x, Wq, Wo, kt, vt, my)


# device time: 293527 ns/iter; 1.0203x vs baseline; 1.0203x over previous
import numpy as np

import jax
import jax.numpy as jnp
from jax import lax
from jax.experimental import pallas as pl
from jax.experimental.pallas import tpu as pltpu

N_DEV = 4
SQ = 2048
SKV = 2048
DM = 1024
HQ_TOTAL = 32
HQ_PER = 8
DH = 128
SCALE = 0.08838834764831843

QBLK = 256
N_QB = SQ // QBLK
HP = 4
BAND = 512
GLOB = 128
NEG = -1e9
KCHUNK = 256

BF16 = jnp.bfloat16


def _kv_headmajor(k_ext, v_ext, my):

    def body(s_ref, k_in, v_in, kt_ref, vt_ref):
        kt_ref[...] = k_in[0].transpose(1, 0, 2).astype(BF16)
        vt_ref[...] = v_in[0].transpose(1, 0, 2).astype(BF16)

    grid_spec = pltpu.PrefetchScalarGridSpec(
        num_scalar_prefetch=1,
        grid=(SKV // KCHUNK, HQ_TOTAL // HQ_PER),
        in_specs=[
            pl.BlockSpec(
                (1, KCHUNK, HQ_PER, DH), lambda sk, j, s: (s[0], sk, j, 0)
            ),
            pl.BlockSpec(
                (1, KCHUNK, HQ_PER, DH), lambda sk, j, s: (s[0], sk, j, 0)
            ),
        ],
        out_specs=[
            pl.BlockSpec((HQ_PER, KCHUNK, DH), lambda sk, j, s: (j, sk, 0)),
            pl.BlockSpec((HQ_PER, KCHUNK, DH), lambda sk, j, s: (j, sk, 0)),
        ],
    )
    return pl.pallas_call(
        body,
        grid_spec=grid_spec,
        out_shape=[
            jax.ShapeDtypeStruct((HQ_TOTAL, SKV, DH), BF16),
            jax.ShapeDtypeStruct((HQ_TOTAL, SKV, DH), BF16),
        ],
        compiler_params=pltpu.CompilerParams(
            dimension_semantics=("arbitrary", "arbitrary"),
        ),
    )(jnp.reshape(my, (1,)).astype(jnp.int32), k_ext, v_ext)


def _band_mask() -> np.ndarray:
    out = np.empty((N_QB, QBLK, GLOB + BAND), np.float32)
    for qb in range(N_QB):
        kstart = int(np.clip(qb * QBLK - 128, GLOB, SKV - BAND))
        qi = (qb * QBLK + np.arange(QBLK))[:, None]
        ki = np.concatenate([np.arange(GLOB), kstart + np.arange(BAND)])[None, :]
        keep = (np.abs(qi - ki) <= 128) | (ki < 32) | (qi < 32)
        out[qb] = np.where(keep, 0.0, NEG)
    return out


def _fused(x, wq_my, wo_my, kt, vt, my):
    def body(s_ref, x_ref, wq_in, wo_in, k_ref, v_ref, mask_ref, o_ref,
             w_scr, ctx_scr, send_sems, recv_sems):
        me = s_ref[0]
        t = pl.program_id(0)
        hh = pl.program_id(1)
        qb = pl.program_id(2)
        right = lax.rem(me + 1, N_DEV)
        left = lax.rem(me + N_DEV - 1, N_DEV)
        g = lax.rem(me + N_DEV - t, N_DEV)

        @pl.when(jnp.logical_and(t == 0, jnp.logical_and(qb == 0, hh == 0)))
        def _first():
            barrier = pltpu.get_barrier_semaphore()
            for nbr in (left, right):
                pl.semaphore_signal(
                    barrier, inc=1, device_id=(nbr,),
                    device_id_type=pl.DeviceIdType.MESH,
                )
            pl.semaphore_wait(barrier, 2)
            w_scr[pl.ds(me, 1), pl.ds(0, 1)] = (
                (wq_in[...] * SCALE).reshape(1, 1, DM, DM).astype(BF16)
            )
            w_scr[pl.ds(me, 1), pl.ds(1, 1)] = (
                wo_in[...].reshape(1, 1, DM, DM).astype(BF16)
            )
            hop1 = pltpu.make_async_remote_copy(
                src_ref=w_scr.at[me],
                dst_ref=w_scr.at[me],
                send_sem=send_sems.at[0],
                recv_sem=recv_sems.at[0],
                device_id=(right,),
                device_id_type=pl.DeviceIdType.MESH,
            )
            hop1.start()

        for k in (1, 2, 3):
            @pl.when(jnp.logical_and(t == k, jnp.logical_and(qb == 0, hh == 0)))
            def _boundary(k=k):
                sent = lax.rem(me + N_DEV - (k - 1), N_DEV)
                got = lax.rem(me + N_DEV - k, N_DEV)
                prev = pltpu.make_async_remote_copy(
                    src_ref=w_scr.at[sent],
                    dst_ref=w_scr.at[got],
                    send_sem=send_sems.at[k - 1],
                    recv_sem=recv_sems.at[k - 1],
                    device_id=(right,),
                    device_id_type=pl.DeviceIdType.MESH,
                )
                prev.wait()
                if k < 3:
                    nxt = pltpu.make_async_remote_copy(
                        src_ref=w_scr.at[got],
                        dst_ref=w_scr.at[got],
                        send_sem=send_sems.at[k],
                        recv_sem=recv_sems.at[k],
                        device_id=(right,),
                        device_id_type=pl.DeviceIdType.MESH,
                    )
                    nxt.start()

        xq = x_ref[0].astype(BF16)
        wq2 = w_scr[g, 0, :, pl.ds(hh * (HP * DH), HP * DH)]
        q2 = jnp.dot(
            xq, wq2, preferred_element_type=jnp.float32
        ).astype(BF16)

        kstart = pl.multiple_of(
            jnp.clip(qb * QBLK - 128, GLOB, SKV - BAND), 128
        )
        for i in range(HP):
            qh = q2[:, i * DH:(i + 1) * DH]
            kb = k_ref[i, pl.ds(kstart, BAND), :]
            vb = v_ref[i, pl.ds(kstart, BAND), :]
            kg = k_ref[i, 0:GLOB, :]
            vg = v_ref[i, 0:GLOB, :]

            sb = lax.dot_general(
                qh, kb, (((1,), (1,)), ((), ())),
                preferred_element_type=jnp.float32,
            ) + mask_ref[0, :, GLOB:]
            sg = lax.dot_general(
                qh, kg, (((1,), (1,)), ((), ())),
                preferred_element_type=jnp.float32,
            ) + mask_ref[0, :, 0:GLOB]

            eb = jnp.exp(sb)
            eg = jnp.exp(sg)
            inv = 1.0 / (
                jnp.sum(eb, axis=-1, keepdims=True)
                + jnp.sum(eg, axis=-1, keepdims=True)
            )
            ctx = (
                jnp.dot(eb.astype(BF16), vb, preferred_element_type=jnp.float32)
                + jnp.dot(eg.astype(BF16), vg, preferred_element_type=jnp.float32)
            ) * inv
            ctx_scr[:, i * DH:(i + 1) * DH] = ctx

            @pl.when(qb == 0)
            def _glob_rows(i=i, qh=qh):
                q32 = qh[0:32]
                s32 = lax.dot_general(
                    q32, k_ref[i], (((1,), (1,)), ((), ())),
                    preferred_element_type=jnp.float32,
                )
                e32 = jnp.exp(s32)
                ctx_scr[0:32, i * DH:(i + 1) * DH] = jnp.dot(
                    e32.astype(BF16), v_ref[i],
                    preferred_element_type=jnp.float32,
                ) * (1.0 / jnp.sum(e32, axis=-1, keepdims=True))

        wo2 = w_scr[g, 1, pl.ds(hh * (HP * DH), HP * DH), :]
        contrib = jnp.dot(
            ctx_scr[...].astype(BF16), wo2, preferred_element_type=jnp.float32
        )
        o_ref[0] = contrib.astype(BF16)

    n_hh = HQ_PER // HP
    grid = (N_DEV, n_hh, N_QB)
    grid_spec = pltpu.PrefetchScalarGridSpec(
        num_scalar_prefetch=1,
        grid=grid,
        in_specs=[
            pl.BlockSpec((1, QBLK, DM), lambda t, hh, qb, s: (0, qb, 0)),
            pl.BlockSpec(memory_space=pltpu.VMEM),
            pl.BlockSpec(memory_space=pltpu.VMEM),
            pl.BlockSpec(
                (HP, SKV, DH),
                lambda t, hh, qb, s: (
                    lax.rem(s[0] + N_DEV - t, N_DEV) * n_hh + hh, 0, 0
                ),
            ),
            pl.BlockSpec(
                (HP, SKV, DH),
                lambda t, hh, qb, s: (
                    lax.rem(s[0] + N_DEV - t, N_DEV) * n_hh + hh, 0, 0
                ),
            ),
            pl.BlockSpec(
                (1, QBLK, GLOB + BAND), lambda t, hh, qb, s: (qb, 0, 0)
            ),
        ],
        out_specs=pl.BlockSpec(
            (1, QBLK, DM), lambda t, hh, qb, s: (t * n_hh + hh, qb, 0)
        ),
        scratch_shapes=[
            pltpu.VMEM((N_DEV, 2, DM, DM), BF16),
            pltpu.VMEM((QBLK, HP * DH), jnp.float32),
            pltpu.SemaphoreType.DMA((N_DEV - 1,)),
            pltpu.SemaphoreType.DMA((N_DEV - 1,)),
        ],
    )
    partials = pl.pallas_call(
        body,
        grid_spec=grid_spec,
        out_shape=jax.ShapeDtypeStruct((N_DEV * n_hh, SQ, DM), BF16),
        compiler_params=pltpu.CompilerParams(
            dimension_semantics=("arbitrary", "arbitrary", "arbitrary"),
            collective_id=0,
        ),
    )(
        jnp.reshape(my, (1,)).astype(jnp.int32),
        x, wq_my, wo_my, kt, vt, jnp.asarray(_band_mask()),
    )
    return jnp.sum(partials, axis=0, keepdims=True, dtype=jnp.float32)


def kernel(x, Wq, K_ext, V_ext, Wo):
    my = lax.axis_index("i")
    kt, vt = _kv_headmajor(K_ext, V_ext, my)
    return _fused(x, Wq, Wo, kt, vt, my)
